# baseline (speedup 1.0000x reference)
import jax
import jax.numpy as jnp
from jax import lax
from jax.experimental import pallas as pl
from jax.experimental.pallas import tpu as pltpu
from jax.experimental.pallas import tpu_sc as plsc

N = 10000
E = 160000
D = 256
DH = 128
K = 96
NTILES = 16
NCH = 106
EPT = NCH * K
E_PAD = NTILES * EPT
E_ALLOC = E_PAD + K
N_ACC = 10240
ROWS_PER_TILE = N_ACC // NTILES
HSEG = ROWS_PER_TILE // 2


def _make_sc_agg(with_deg):
    def body(*refs):
        if with_deg:
            (table, srcp, dstp, zrows, out_sum, out_deg,
             acc_sp, stage_sp, src_a, src_b, dst_a, dst_b, idx2_a, idx2_b,
             rows_a, rows_b, deg_loc, deg640, tmp640,
             semg_a, semg_b, sems_a, sems_b) = refs
        else:
            (table, srcp, dstp, zrows, out_sum,
             acc_sp, src_a, src_b, dst_a, dst_b, idx2_a, idx2_b,
             rows_a, rows_b,
             semg_a, semg_b, sems_a, sems_b) = refs

        c = lax.axis_index("c")
        s = lax.axis_index("s")
        zero16 = jnp.zeros((16,), jnp.float32)
        ones16 = jnp.ones((16,), jnp.float32)
        trash16 = jnp.full((16,), N, jnp.int32)

        def load_idx(g, srcb, dstb, idx2b):
            off = g * K
            pltpu.sync_copy(srcp.at[pl.ds(off, K)], srcb)
            pltpu.sync_copy(dstp.at[pl.ds(off, K)], dstb.at[0])
            for i in range(K // 16):
                sl = pl.ds(i * 16, 16)
                idx2b[sl] = srcb[sl] * 2 + c

        def start_gather(idx2b, rowsb, sem):
            pltpu.async_copy(table.at[idx2b], rowsb, sem)

        def wait_gather(idx2b, rowsb, sem):
            pltpu.make_async_copy(table.at[idx2b], rowsb, sem).wait()

        def start_scatter(rowsb, dstb, sem):
            pltpu.async_copy(rowsb, acc_sp.at[dstb.at[0]], sem, add=True)

        def wait_scatter(rowsb, dstb, sem):
            pltpu.make_async_copy(rowsb, acc_sp.at[dstb.at[0]], sem).wait()

        def deg_add(dstb):
            if with_deg:
                for i in range(K // 16):
                    sl = pl.ds(i * 16, 16)
                    plsc.addupdate_scatter(deg_loc, [dstb[0, sl]], ones16)

        r0 = pl.multiple_of(s * ROWS_PER_TILE, ROWS_PER_TILE)
        co = pl.multiple_of(c * DH, DH)
        pltpu.sync_copy(zrows, rows_a)
        for k in range(ROWS_PER_TILE // K):
            pltpu.sync_copy(rows_a, acc_sp.at[pl.ds(r0 + k * K, K), :])
        rem = ROWS_PER_TILE % K
        if rem:
            pltpu.sync_copy(
                rows_a.at[pl.ds(0, rem)],
                acc_sp.at[pl.ds(r0 + (ROWS_PER_TILE // K) * K, rem), :])
        if with_deg:
            def zbody(i, carry):
                deg_loc[pl.ds(i * 16, 16)] = zero16
                return carry

            lax.fori_loop(0, N_ACC // 16, zbody, 0)

        plsc.subcore_barrier()

        base_ch = s * NCH

        for i in range(K // 16):
            dst_b[0, pl.ds(i * 16, 16)] = trash16
        start_scatter(rows_b, dst_b, sems_b)
        load_idx(base_ch, src_a, dst_a, idx2_a)
        start_gather(idx2_a, rows_a, semg_a)

        def pair(j, carry):
            a = base_ch + 2 * j
            wait_gather(idx2_a, rows_a, semg_a)
            start_scatter(rows_a, dst_a, sems_a)
            deg_add(dst_a)
            wait_scatter(rows_b, dst_b, sems_b)
            load_idx(a + 1, src_b, dst_b, idx2_b)
            start_gather(idx2_b, rows_b, semg_b)
            wait_gather(idx2_b, rows_b, semg_b)
            start_scatter(rows_b, dst_b, sems_b)
            deg_add(dst_b)
            wait_scatter(rows_a, dst_a, sems_a)
            load_idx(a + 2, src_a, dst_a, idx2_a)
            start_gather(idx2_a, rows_a, semg_a)
            return carry

        lax.fori_loop(0, NCH // 2, pair, 0)

        wait_scatter(rows_b, dst_b, sems_b)
        wait_gather(idx2_a, rows_a, semg_a)

        if with_deg:
            pltpu.sync_copy(deg_loc, stage_sp.at[s])
            plsc.subcore_barrier()

            def z640(i, carry):
                deg640[pl.ds(i * 16, 16)] = zero16
                return carry

            lax.fori_loop(0, ROWS_PER_TILE // 16, z640, 0)
            for t in range(NTILES):
                pltpu.sync_copy(stage_sp.at[t, pl.ds(r0, ROWS_PER_TILE)],
                                tmp640)

                def a640(i, carry):
                    sl = pl.ds(i * 16, 16)
                    deg640[sl] = deg640[sl] + tmp640[sl]
                    return carry

                lax.fori_loop(0, ROWS_PER_TILE // 16, a640, 0)
            o0 = pl.multiple_of(s * ROWS_PER_TILE + c * HSEG, HSEG)
            pltpu.sync_copy(deg640.at[pl.ds(c * HSEG, HSEG)],
                            out_deg.at[pl.ds(o0, HSEG)])

        plsc.subcore_barrier()

        for k in range(ROWS_PER_TILE // K):
            rr = r0 + k * K
            pltpu.sync_copy(acc_sp.at[pl.ds(rr, K), :], rows_a)
            pltpu.sync_copy(rows_a, out_sum.at[pl.ds(rr, K), pl.ds(co, DH)])
        if rem:
            rr = r0 + (ROWS_PER_TILE // K) * K
            pltpu.sync_copy(acc_sp.at[pl.ds(rr, rem), :],
                            rows_a.at[pl.ds(0, rem)])
            pltpu.sync_copy(rows_a.at[pl.ds(0, rem)],
                            out_sum.at[pl.ds(rr, rem), pl.ds(co, DH)])

    if with_deg:
        out_type = (jax.ShapeDtypeStruct((N_ACC, D), jnp.float32),
                    jax.ShapeDtypeStruct((N_ACC,), jnp.float32))
    else:
        out_type = jax.ShapeDtypeStruct((N_ACC, D), jnp.float32)
    scratch = [pltpu.VMEM_SHARED((N_ACC, DH), jnp.float32)]
    if with_deg:
        scratch.append(pltpu.VMEM_SHARED((NTILES, N_ACC), jnp.float32))
    scratch += [
        pltpu.VMEM((K,), jnp.int32),
        pltpu.VMEM((K,), jnp.int32),
        pltpu.VMEM((1, K), jnp.int32),
        pltpu.VMEM((1, K), jnp.int32),
        pltpu.VMEM((K,), jnp.int32),
        pltpu.VMEM((K,), jnp.int32),
        pltpu.VMEM((K, DH), jnp.float32),
        pltpu.VMEM((K, DH), jnp.float32),
    ]
    if with_deg:
        scratch.append(pltpu.VMEM((N_ACC,), jnp.float32))
        scratch.append(pltpu.VMEM((ROWS_PER_TILE,), jnp.float32))
        scratch.append(pltpu.VMEM((ROWS_PER_TILE,), jnp.float32))
    scratch += [pltpu.SemaphoreType.DMA] * 4
    return pl.kernel(
        body,
        out_type=out_type,
        mesh=plsc.VectorSubcoreMesh(core_axis_name="c", subcore_axis_name="s"),
        scratch_types=scratch,
        compiler_params=pltpu.CompilerParams(needs_layout_passes=False),
    )


_sc_agg_deg = _make_sc_agg(with_deg=True)
_sc_agg = _make_sc_agg(with_deg=False)


BLK = 1000


def _tc1_body(deg_ref, s_ref, x_ref, wl_ref, wr_ref, bl_ref, o_ref):
    r = 1.0 / jnp.maximum(deg_ref[...], 1.0)
    mean = s_ref[...] * r
    acc = lax.dot_general(mean, wl_ref[...], (((1,), (1,)), ((), ())),
                          preferred_element_type=jnp.float32)
    acc = acc + lax.dot_general(x_ref[...], wr_ref[...], (((1,), (1,)), ((), ())),
                                preferred_element_type=jnp.float32)
    o_ref[...] = jnp.maximum(acc + bl_ref[...], 0.0)


def _tc2_body(deg_ref, s_ref, x1_ref, wl_ref, wr_ref, bl_ref, wlin_ref,
              blin_ref, o_ref):
    r = 1.0 / jnp.maximum(deg_ref[...], 1.0)
    mean = s_ref[...] * r
    acc = lax.dot_general(mean, wl_ref[...], (((1,), (1,)), ((), ())),
                          preferred_element_type=jnp.float32)
    acc = acc + lax.dot_general(x1_ref[...], wr_ref[...], (((1,), (1,)), ((), ())),
                                preferred_element_type=jnp.float32)
    x2 = jnp.maximum(acc + bl_ref[...], 0.0)
    z = lax.dot_general(x1_ref[...], wlin_ref[:, :D], (((1,), (1,)), ((), ())),
                        preferred_element_type=jnp.float32)
    z = z + lax.dot_general(x2, wlin_ref[:, D:], (((1,), (1,)), ((), ())),
                            preferred_element_type=jnp.float32)
    o_ref[...] = jax.nn.sigmoid(z + blin_ref[...])


def _tc_layer1(deg16, s, x, Wl, Wr, bl):
    return pl.pallas_call(
        _tc1_body,
        grid=(N // BLK,),
        in_specs=[
            pl.BlockSpec((BLK, 1), lambda i: (i, 0)),
            pl.BlockSpec((BLK, D), lambda i: (i, 0)),
            pl.BlockSpec((BLK, D), lambda i: (i, 0)),
            pl.BlockSpec((D, D), lambda i: (0, 0)),
            pl.BlockSpec((D, D), lambda i: (0, 0)),
            pl.BlockSpec((1, D), lambda i: (0, 0)),
        ],
        out_specs=pl.BlockSpec((BLK, D), lambda i: (i, 0)),
        out_shape=jax.ShapeDtypeStruct((N, D), jnp.float32),
    )(deg16, s, x, Wl, Wr, bl)


def _tc_layer2(deg16, s, x1, Wl, Wr, bl, Wlin, blin):
    return pl.pallas_call(
        _tc2_body,
        grid=(N // BLK,),
        in_specs=[
            pl.BlockSpec((BLK, 1), lambda i: (i, 0)),
            pl.BlockSpec((BLK, D), lambda i: (i, 0)),
            pl.BlockSpec((BLK, D), lambda i: (i, 0)),
            pl.BlockSpec((D, D), lambda i: (0, 0)),
            pl.BlockSpec((D, D), lambda i: (0, 0)),
            pl.BlockSpec((1, D), lambda i: (0, 0)),
            pl.BlockSpec((1, 2 * D), lambda i: (0, 0)),
            pl.BlockSpec((1, 1), lambda i: (0, 0)),
        ],
        out_specs=pl.BlockSpec((BLK, 1), lambda i: (i, 0)),
        out_shape=jax.ShapeDtypeStruct((N, 1), jnp.float32),
    )(deg16, s, x1, Wl, Wr, bl, Wlin, blin)


def kernel(x, edge_index, Wl1, bl1, Wr1, Wl2, bl2, Wr2, Wlin, blin):
    src = edge_index[0].astype(jnp.int32)
    dst = edge_index[1].astype(jnp.int32)
    pad = E_ALLOC - E
    srcp = jnp.concatenate([src, jnp.zeros((pad,), jnp.int32)])
    dstp = jnp.concatenate([dst, jnp.full((pad,), N, jnp.int32)])
    zrows = jnp.zeros((K, DH), jnp.float32)

    bl1r = bl1.reshape(1, D)
    bl2r = bl2.reshape(1, D)
    blinr = blin.reshape(1, 1)

    sum1, deg = _sc_agg_deg(x.reshape(2 * N, DH), srcp, dstp, zrows)
    degc = deg.reshape(N_ACC, 1)
    x1 = _tc_layer1(degc, sum1, x, Wl1, Wr1, bl1r)
    sum2 = _sc_agg(x1.reshape(2 * N, DH), srcp, dstp, zrows)
    return _tc_layer2(degc, sum2, x1, Wl2, Wr2, bl2r, Wlin, blinr)


# speedup vs baseline: 1.1799x; 1.1799x over previous
import jax
import jax.numpy as jnp
from jax import lax
from jax.experimental import pallas as pl
from jax.experimental.pallas import tpu as pltpu
from jax.experimental.pallas import tpu_sc as plsc

N = 10000
E = 160000
D = 256
DH = 128
K = 128
NTILES = 16
NCH = 80
EPT = NCH * K
E_PAD = NTILES * EPT
E_ALLOC = E_PAD + 2 * K
NCH_ALL = E_ALLOC // K
N_ACC = 10240
ROWS_PER_TILE = N_ACC // NTILES
HSEG = ROWS_PER_TILE // 2


def _make_sc_agg(with_deg):
    def body(*refs):
        if with_deg:
            (table, ei, zrows, out_sum, out_deg, out_stage,
             acc_sp, ei_a, ei_b, idx2_a, idx2_b, dsts_a, dsts_b,
             rows_a, rows_b, deg_loc, deg640, tmp640,
             semg_a, semg_b, sems_a, sems_b, semi_a, semi_b) = refs
        else:
            (table, ei, zrows, out_sum,
             acc_sp, ei_a, ei_b, idx2_a, idx2_b, dsts_a, dsts_b,
             rows_a, rows_b,
             semg_a, semg_b, sems_a, sems_b, semi_a, semi_b) = refs

        c = lax.axis_index("c")
        s = lax.axis_index("s")
        zero16 = jnp.zeros((16,), jnp.float32)
        ones16 = jnp.ones((16,), jnp.float32)
        trash16 = jnp.full((16,), N, jnp.int32)

        def compute_idx2(eib, idx2b):
            for i in range(K // 16):
                sl = pl.ds(i * 16, 16)
                idx2b[sl] = eib[0, sl] * 2 + c

        def copy_dst(eib, dstsb):
            for i in range(K // 16):
                sl = pl.ds(i * 16, 16)
                dstsb[0, sl] = eib[1, sl]

        def start_gather(idx2b, rowsb, sem):
            pltpu.async_copy(table.at[idx2b], rowsb, sem)

        def wait_gather(idx2b, rowsb, sem):
            pltpu.make_async_copy(table.at[idx2b], rowsb, sem).wait()

        def start_scatter(rowsb, dstsb, sem):
            pltpu.async_copy(rowsb, acc_sp.at[dstsb.at[0]], sem, add=True)

        def wait_scatter(rowsb, dstsb, sem):
            pltpu.make_async_copy(rowsb, acc_sp.at[dstsb.at[0]], sem).wait()

        def start_ei(g, eib, sem):
            pltpu.async_copy(ei.at[g], eib, sem)

        def wait_ei(g, eib, sem):
            pltpu.make_async_copy(ei.at[g], eib, sem).wait()

        def deg_add(dstsb):
            if with_deg:
                for i in range(K // 16):
                    sl = pl.ds(i * 16, 16)
                    plsc.addupdate_scatter(deg_loc, [dstsb[0, sl]], ones16)

        r0 = pl.multiple_of(s * ROWS_PER_TILE, ROWS_PER_TILE)
        co = pl.multiple_of(c * DH, DH)
        pltpu.sync_copy(zrows, rows_a)
        for k in range(ROWS_PER_TILE // K):
            pltpu.sync_copy(rows_a, acc_sp.at[pl.ds(r0 + k * K, K), :])
        if with_deg:
            def zbody(i, carry):
                deg_loc[pl.ds(i * 16, 16)] = zero16
                return carry

            lax.fori_loop(0, N_ACC // 16, zbody, 0)

        plsc.subcore_barrier()

        base_ch = s * NCH

        for i in range(K // 16):
            dsts_b[0, pl.ds(i * 16, 16)] = trash16
        start_scatter(rows_b, dsts_b, sems_b)
        pltpu.sync_copy(ei.at[base_ch], ei_a)
        compute_idx2(ei_a, idx2_a)
        copy_dst(ei_a, dsts_a)
        start_gather(idx2_a, rows_a, semg_a)
        start_ei(base_ch + 1, ei_b, semi_b)

        def pair(j, carry):
            a = base_ch + 2 * j
            b = a + 1
            wait_ei(b, ei_b, semi_b)
            compute_idx2(ei_b, idx2_b)
            wait_scatter(rows_b, dsts_b, sems_b)
            copy_dst(ei_b, dsts_b)
            start_gather(idx2_b, rows_b, semg_b)
            start_ei(a + 2, ei_a, semi_a)
            wait_gather(idx2_a, rows_a, semg_a)
            start_scatter(rows_a, dsts_a, sems_a)
            deg_add(dsts_a)
            wait_ei(a + 2, ei_a, semi_a)
            compute_idx2(ei_a, idx2_a)
            wait_scatter(rows_a, dsts_a, sems_a)
            copy_dst(ei_a, dsts_a)
            start_gather(idx2_a, rows_a, semg_a)
            start_ei(b + 2, ei_b, semi_b)
            wait_gather(idx2_b, rows_b, semg_b)
            start_scatter(rows_b, dsts_b, sems_b)
            deg_add(dsts_b)
            return carry

        lax.fori_loop(0, NCH // 2, pair, 0)

        wait_scatter(rows_b, dsts_b, sems_b)
        wait_gather(idx2_a, rows_a, semg_a)
        wait_ei(base_ch + NCH + 1, ei_b, semi_b)

        if with_deg:
            so = pl.multiple_of((c * NTILES + s) * N_ACC, N_ACC)
            pltpu.sync_copy(deg_loc, out_stage.at[pl.ds(so, N_ACC)])
            plsc.subcore_barrier()

            def z640(i, carry):
                deg640[pl.ds(i * 16, 16)] = zero16
                return carry

            lax.fori_loop(0, ROWS_PER_TILE // 16, z640, 0)
            for t in range(NTILES):
                to = pl.multiple_of((c * NTILES + t) * N_ACC + r0,
                                    ROWS_PER_TILE)
                pltpu.sync_copy(out_stage.at[pl.ds(to, ROWS_PER_TILE)],
                                tmp640)

                def a640(i, carry):
                    sl = pl.ds(i * 16, 16)
                    deg640[sl] = deg640[sl] + tmp640[sl]
                    return carry

                lax.fori_loop(0, ROWS_PER_TILE // 16, a640, 0)
            o0 = pl.multiple_of(s * ROWS_PER_TILE + c * HSEG, HSEG)
            pltpu.sync_copy(deg640.at[pl.ds(c * HSEG, HSEG)],
                            out_deg.at[pl.ds(o0, HSEG)])

        plsc.subcore_barrier()

        for k in range(ROWS_PER_TILE // K):
            rr = r0 + k * K
            pltpu.sync_copy(acc_sp.at[pl.ds(rr, K), :], rows_a)
            pltpu.sync_copy(rows_a, out_sum.at[pl.ds(rr, K), pl.ds(co, DH)])

    if with_deg:
        out_type = (jax.ShapeDtypeStruct((N_ACC, D), jnp.float32),
                    jax.ShapeDtypeStruct((N_ACC,), jnp.float32),
                    jax.ShapeDtypeStruct((2 * NTILES * N_ACC,), jnp.float32))
    else:
        out_type = jax.ShapeDtypeStruct((N_ACC, D), jnp.float32)
    scratch = [
        pltpu.VMEM_SHARED((N_ACC, DH), jnp.float32),
        pltpu.VMEM((2, K), jnp.int32),
        pltpu.VMEM((2, K), jnp.int32),
        pltpu.VMEM((K,), jnp.int32),
        pltpu.VMEM((K,), jnp.int32),
        pltpu.VMEM((1, K), jnp.int32),
        pltpu.VMEM((1, K), jnp.int32),
        pltpu.VMEM((K, DH), jnp.float32),
        pltpu.VMEM((K, DH), jnp.float32),
    ]
    if with_deg:
        scratch.append(pltpu.VMEM((N_ACC,), jnp.float32))
        scratch.append(pltpu.VMEM((ROWS_PER_TILE,), jnp.float32))
        scratch.append(pltpu.VMEM((ROWS_PER_TILE,), jnp.float32))
    scratch += [pltpu.SemaphoreType.DMA] * 6
    return pl.kernel(
        body,
        out_type=out_type,
        mesh=plsc.VectorSubcoreMesh(core_axis_name="c", subcore_axis_name="s"),
        scratch_types=scratch,
        compiler_params=pltpu.CompilerParams(needs_layout_passes=False),
    )


_sc_agg_deg = _make_sc_agg(with_deg=True)
_sc_agg = _make_sc_agg(with_deg=False)


BLK = 1000


def _tc1_body(deg_ref, s_ref, x_ref, wl_ref, wr_ref, bl_ref, o_ref):
    r = 1.0 / jnp.maximum(deg_ref[...], 1.0)
    mean = s_ref[...] * r
    acc = lax.dot_general(mean, wl_ref[...], (((1,), (1,)), ((), ())),
                          preferred_element_type=jnp.float32)
    acc = acc + lax.dot_general(x_ref[...], wr_ref[...], (((1,), (1,)), ((), ())),
                                preferred_element_type=jnp.float32)
    o_ref[...] = jnp.maximum(acc + bl_ref[...], 0.0)


def _tc2_body(deg_ref, s_ref, x1_ref, wl_ref, wr_ref, bl_ref, wlin_ref,
              blin_ref, o_ref):
    r = 1.0 / jnp.maximum(deg_ref[...], 1.0)
    mean = s_ref[...] * r
    acc = lax.dot_general(mean, wl_ref[...], (((1,), (1,)), ((), ())),
                          preferred_element_type=jnp.float32)
    acc = acc + lax.dot_general(x1_ref[...], wr_ref[...], (((1,), (1,)), ((), ())),
                                preferred_element_type=jnp.float32)
    x2 = jnp.maximum(acc + bl_ref[...], 0.0)
    z = lax.dot_general(x1_ref[...], wlin_ref[:, :D], (((1,), (1,)), ((), ())),
                        preferred_element_type=jnp.float32)
    z = z + lax.dot_general(x2, wlin_ref[:, D:], (((1,), (1,)), ((), ())),
                            preferred_element_type=jnp.float32)
    o_ref[...] = jax.nn.sigmoid(z + blin_ref[...])


def _tc_layer1(deg16, s, x, Wl, Wr, bl):
    return pl.pallas_call(
        _tc1_body,
        grid=(N // BLK,),
        in_specs=[
            pl.BlockSpec((BLK, 1), lambda i: (i, 0)),
            pl.BlockSpec((BLK, D), lambda i: (i, 0)),
            pl.BlockSpec((BLK, D), lambda i: (i, 0)),
            pl.BlockSpec((D, D), lambda i: (0, 0)),
            pl.BlockSpec((D, D), lambda i: (0, 0)),
            pl.BlockSpec((1, D), lambda i: (0, 0)),
        ],
        out_specs=pl.BlockSpec((BLK, D), lambda i: (i, 0)),
        out_shape=jax.ShapeDtypeStruct((N, D), jnp.float32),
    )(deg16, s, x, Wl, Wr, bl)


def _tc_layer2(deg16, s, x1, Wl, Wr, bl, Wlin, blin):
    return pl.pallas_call(
        _tc2_body,
        grid=(N // BLK,),
        in_specs=[
            pl.BlockSpec((BLK, 1), lambda i: (i, 0)),
            pl.BlockSpec((BLK, D), lambda i: (i, 0)),
            pl.BlockSpec((BLK, D), lambda i: (i, 0)),
            pl.BlockSpec((D, D), lambda i: (0, 0)),
            pl.BlockSpec((D, D), lambda i: (0, 0)),
            pl.BlockSpec((1, D), lambda i: (0, 0)),
            pl.BlockSpec((1, 2 * D), lambda i: (0, 0)),
            pl.BlockSpec((1, 1), lambda i: (0, 0)),
        ],
        out_specs=pl.BlockSpec((BLK, 1), lambda i: (i, 0)),
        out_shape=jax.ShapeDtypeStruct((N, 1), jnp.float32),
    )(deg16, s, x1, Wl, Wr, bl, Wlin, blin)


def kernel(x, edge_index, Wl1, bl1, Wr1, Wl2, bl2, Wr2, Wlin, blin):
    src = edge_index[0].astype(jnp.int32)
    dst = edge_index[1].astype(jnp.int32)
    pad = E_ALLOC - E
    srcp = jnp.concatenate([src, jnp.zeros((pad,), jnp.int32)])
    dstp = jnp.concatenate([dst, jnp.full((pad,), N, jnp.int32)])
    ei = jnp.stack([srcp.reshape(NCH_ALL, K), dstp.reshape(NCH_ALL, K)],
                   axis=1)
    zrows = jnp.zeros((K, DH), jnp.float32)

    bl1r = bl1.reshape(1, D)
    bl2r = bl2.reshape(1, D)
    blinr = blin.reshape(1, 1)

    sum1, deg, _stage = _sc_agg_deg(x.reshape(2 * N, DH), ei, zrows)
    degc = deg.reshape(N_ACC, 1)
    x1 = _tc_layer1(degc, sum1, x, Wl1, Wr1, bl1r)
    sum2 = _sc_agg(x1.reshape(2 * N, DH), ei, zrows)
    return _tc_layer2(degc, sum2, x1, Wl2, Wr2, bl2r, Wlin, blinr)
